# baseline (speedup 1.0000x reference)
import functools

import jax
import jax.numpy as jnp
from jax import lax
from jax.experimental import pallas as pl
from jax.experimental.pallas import tpu as pltpu
from jax.experimental.pallas import tpu_sc as plsc

BATCH = 4096
NUM_FIELDS = 26
NC = 2
NS = 16
LANES = 16
NW = NC * NS
ROWS_PER_W = BATCH // NW
VOCAB_N = 100000
SLICE = 6256
HALF = SLICE // 2
TAIL_N = VOCAB_N - (NS - 1) * SLICE - HALF
F_G0 = 13


def _sc_kernel(xt_hbm, table_hbm, bias_hbm, out_hbm, spt, idx_v, vals_v,
               out_v, bias_v, tab_v, semi, semb, semta, semtb, semt2,
               semg0, semg1):
    cid = lax.axis_index("c")
    sid = lax.axis_index("s")
    wid = sid * NC + cid
    base = wid * ROWS_PER_W

    stage = [
        pltpu.async_copy(
            xt_hbm.at[f, pl.ds(base, ROWS_PER_W)],
            idx_v.at[pl.ds(f * ROWS_PER_W, ROWS_PER_W)],
            semi,
        )
        for f in range(NUM_FIELDS)
    ]
    bias_cp = pltpu.async_copy(bias_hbm, bias_v, semb)

    slice_start = sid * SLICE
    h1a = pltpu.async_copy(table_hbm.at[pl.ds(slice_start, HALF)],
                           tab_v.at[pl.ds(0, HALF)], semta)

    @pl.when(sid != NS - 1)
    def _issue_h1b_full():
        pltpu.async_copy(table_hbm.at[pl.ds(slice_start + HALF, HALF)],
                         tab_v.at[pl.ds(HALF, HALF)], semtb)

    @pl.when(sid == NS - 1)
    def _issue_h1b_tail():
        pltpu.async_copy(table_hbm.at[pl.ds(slice_start + HALF, TAIL_N)],
                         tab_v.at[pl.ds(HALF, TAIL_N)], semtb)

    h1a.wait()
    h2a = pltpu.async_copy(tab_v.at[pl.ds(0, HALF)],
                           spt.at[pl.ds(slice_start, HALF)], semt2)

    @pl.when(sid != NS - 1)
    def _bounce_h2b_full():
        pltpu.make_async_copy(table_hbm.at[pl.ds(slice_start + HALF, HALF)],
                              tab_v.at[pl.ds(HALF, HALF)], semtb).wait()
        pltpu.sync_copy(tab_v.at[pl.ds(HALF, HALF)],
                        spt.at[pl.ds(slice_start + HALF, HALF)])

    @pl.when(sid == NS - 1)
    def _bounce_h2b_tail():
        pltpu.make_async_copy(table_hbm.at[pl.ds(slice_start + HALF, TAIL_N)],
                              tab_v.at[pl.ds(HALF, TAIL_N)], semtb).wait()
        pltpu.sync_copy(tab_v.at[pl.ds(HALF, TAIL_N)],
                        spt.at[pl.ds(slice_start + HALF, TAIL_N)])

    h2a.wait()
    plsc.subcore_barrier()

    for cp in stage:
        cp.wait()

    n0 = F_G0 * ROWS_PER_W
    n1 = (NUM_FIELDS - F_G0) * ROWS_PER_W
    g0 = pltpu.async_copy(spt.at[idx_v.at[pl.ds(0, n0)]],
                          vals_v.at[pl.ds(0, n0)], semg0)
    g1 = pltpu.async_copy(spt.at[idx_v.at[pl.ds(n0, n1)]],
                          vals_v.at[pl.ds(n0, n1)], semg1)

    bias_cp.wait()
    bias_vec = plsc.load_gather(bias_v, [jnp.zeros((LANES,), jnp.int32)])

    nchunk = ROWS_PER_W // LANES
    accs = [bias_vec] * nchunk
    g0.wait()
    for j in range(nchunk):
        acc = accs[j]
        for f in range(F_G0):
            acc = acc + vals_v[pl.ds(f * ROWS_PER_W + j * LANES, LANES)]
        accs[j] = acc
    g1.wait()
    for j in range(nchunk):
        acc = accs[j]
        for f in range(F_G0, NUM_FIELDS):
            acc = acc + vals_v[pl.ds(f * ROWS_PER_W + j * LANES, LANES)]
        out_v[pl.ds(j * LANES, LANES)] = acc

    pltpu.sync_copy(out_v, out_hbm.at[pl.ds(base, ROWS_PER_W)])


@jax.jit
def _run(xt, table_flat, bias):
    mesh = plsc.VectorSubcoreMesh(
        core_axis_name="c", subcore_axis_name="s",
        num_cores=NC, num_subcores=NS)
    f = functools.partial(
        pl.kernel,
        out_type=jax.ShapeDtypeStruct((BATCH,), jnp.float32),
        mesh=mesh,
        scratch_types=[
            pltpu.VMEM_SHARED((NS * SLICE,), jnp.float32),
            pltpu.VMEM((NUM_FIELDS * ROWS_PER_W,), jnp.int32),
            pltpu.VMEM((NUM_FIELDS * ROWS_PER_W,), jnp.float32),
            pltpu.VMEM((ROWS_PER_W,), jnp.float32),
            pltpu.VMEM((1,), jnp.float32),
            pltpu.VMEM((SLICE,), jnp.float32),
            pltpu.SemaphoreType.DMA,
            pltpu.SemaphoreType.DMA,
            pltpu.SemaphoreType.DMA,
            pltpu.SemaphoreType.DMA,
            pltpu.SemaphoreType.DMA,
            pltpu.SemaphoreType.DMA,
            pltpu.SemaphoreType.DMA,
        ],
        compiler_params=pltpu.CompilerParams(needs_layout_passes=False),
    )(_sc_kernel)
    return f(xt, table_flat, bias)


def kernel(x, table, bias):
    xt = x.astype(jnp.int32).T
    table_flat = table.reshape(-1)
    out = _run(xt, table_flat, bias.astype(jnp.float32))
    return out.reshape(BATCH, 1)
